# baseline (speedup 1.0000x reference)
import functools
import jax
import jax.numpy as jnp
from jax import lax
from jax.experimental import pallas as pl
from jax.experimental.pallas import tpu as pltpu
from jax.experimental.pallas import tpu_sc as plsc

K = 100000
D = 512
C = 64
NN = 64
BK = 2048
NB = (K + BK - 1) // BK
KP = NB * BK


def _dist_topk_kernel(x_ref, q_ref, d_out, i_out, d_ref):
    b = pl.program_id(0)

    xb = x_ref[...]
    q = q_ref[...]
    diff = xb - q[None, :]
    sq = diff * diff
    part = (sq[:, 0:128] + sq[:, 128:256]) + (sq[:, 256:384] + sq[:, 384:512])
    pt = lax.transpose(part, (1, 0))
    dsq = jnp.sum(pt, axis=0, keepdims=True)
    dist = jnp.sqrt(dsq)
    col = lax.broadcasted_iota(jnp.int32, (1, BK), 1)
    valid = (b * BK + col) < K
    dist = jnp.where(valid, dist, jnp.inf)

    d_ref[pl.ds(b, 1), :] = dist

    @pl.when(b == NB - 1)
    def _finish():
        row_i = lax.broadcasted_iota(jnp.int32, (NB, BK), 0)
        col_i = lax.broadcasted_iota(jnp.int32, (NB, BK), 1)
        gidx = row_i * BK + col_i
        iota64 = lax.broadcasted_iota(jnp.int32, (1, NN), 1)
        BIGI = jnp.int32(2147483647)

        def body(t, carry):
            d_top, i_top = carry
            a = d_ref[...]
            m = jnp.min(a)
            idx = jnp.min(jnp.where(a == m, gidx, BIGI))
            hit = gidx == idx
            d_ref[...] = jnp.where(hit, jnp.inf, a)
            d_top = jnp.where(iota64 == t, m, d_top)
            i_top = jnp.where(iota64 == t, idx, i_top)
            return d_top, i_top

        d_top0 = jnp.full((1, NN), jnp.inf, dtype=jnp.float32)
        i_top0 = jnp.zeros((1, NN), dtype=jnp.int32)
        d_top, i_top = lax.fori_loop(0, NN, body, (d_top0, i_top0))
        d_out[...] = d_top
        i_out[...] = i_top


def _dist_topk(input, X):
    return pl.pallas_call(
        _dist_topk_kernel,
        grid=(NB,),
        in_specs=[
            pl.BlockSpec((BK, D), lambda b: (b, 0)),
            pl.BlockSpec((D,), lambda b: (0,)),
        ],
        out_specs=[
            pl.BlockSpec((1, NN), lambda b: (0, 0)),
            pl.BlockSpec((1, NN), lambda b: (0, 0)),
        ],
        out_shape=[
            jax.ShapeDtypeStruct((1, NN), jnp.float32),
            jax.ShapeDtypeStruct((1, NN), jnp.int32),
        ],
        scratch_shapes=[pltpu.VMEM((NB, BK), jnp.float32)],
    )(X, input)


def _sc_gather_kernel(y_hbm, idx_hbm, rows_hbm, idx_v, rows_v, sem):
    cid = lax.axis_index("c")
    sid = lax.axis_index("s")

    @pl.when(jnp.logical_and(cid == 0, sid == 0))
    def _():
        pltpu.sync_copy(idx_hbm, idx_v)
        pltpu.async_copy(y_hbm.at[idx_v], rows_v, sem).wait()
        pltpu.sync_copy(rows_v, rows_hbm)


def _sc_gather(y, idx_top):
    mesh = plsc.VectorSubcoreMesh(core_axis_name="c", subcore_axis_name="s")
    f = pl.kernel(
        _sc_gather_kernel,
        mesh=mesh,
        compiler_params=pltpu.CompilerParams(use_tc_tiling_on_sc=False),
        out_type=jax.ShapeDtypeStruct((NN, C), jnp.float32),
        scratch_types=[
            pltpu.VMEM((NN,), jnp.int32),
            pltpu.VMEM((NN, C), jnp.float32),
            pltpu.SemaphoreType.DMA,
        ],
    )
    return f(y, idx_top)


def _vote_kernel(rows_ref, d_ref, out_ref):
    iota64 = lax.broadcasted_iota(jnp.int32, (1, NN), 1)
    rows = rows_ref[...]
    labs = jnp.sum(rows * iota64.astype(jnp.float32),
                   axis=1, keepdims=True).astype(jnp.int32)
    qv = jnp.float32(1.0) / (-d_ref[...])
    amask = labs == iota64
    S = jnp.sum(jnp.where(amask, qv, jnp.float32(0.0)),
                axis=1, keepdims=True)
    rmax = jnp.max(S)
    iota_col = lax.broadcasted_iota(jnp.int32, (NN, 1), 0)
    r = jnp.min(jnp.where(S == rmax, iota_col, jnp.int32(NN)))
    out_ref[...] = (iota64 == r).astype(jnp.float32).reshape(NN)


def _vote(rows, d_top):
    return pl.pallas_call(
        _vote_kernel,
        out_shape=jax.ShapeDtypeStruct((NN,), jnp.float32),
    )(rows, d_top)


@jax.jit
def kernel(input, X, y):
    d_top, i_top = _dist_topk(input, X)
    rows = _sc_gather(y, i_top.reshape(NN))
    return _vote(rows, d_top)


# speedup vs baseline: 1.3559x; 1.3559x over previous
import jax
import jax.numpy as jnp
from jax import lax
from jax.experimental import pallas as pl
from jax.experimental.pallas import tpu as pltpu

K = 100000
D = 512
C = 64
NN = 64
BK = 2048
NB = (K + BK - 1) // BK
KP = NB * BK


def _knn_kernel(x_ref, q_ref, lab_ref, out_ref, d_ref, l_ref):
    b = pl.program_id(0)

    xb = x_ref[...]
    q = q_ref[...]
    diff = xb - q[None, :]
    sq = diff * diff
    part = (sq[:, 0:128] + sq[:, 128:256]) + (sq[:, 256:384] + sq[:, 384:512])
    pt = lax.transpose(part, (1, 0))
    dsq = jnp.sum(pt, axis=0, keepdims=True)
    dist = jnp.sqrt(dsq)
    col = lax.broadcasted_iota(jnp.int32, (1, BK), 1)
    valid = (b * BK + col) < K
    dist = jnp.where(valid, dist, jnp.inf)

    d_ref[pl.ds(b, 1), :] = dist
    l_ref[pl.ds(b, 1), :] = lab_ref[...].reshape(1, BK)

    @pl.when(b == NB - 1)
    def _finish():
        row_i = lax.broadcasted_iota(jnp.int32, (NB, BK), 0)
        col_i = lax.broadcasted_iota(jnp.int32, (NB, BK), 1)
        gidx = row_i * BK + col_i
        iota64 = lax.broadcasted_iota(jnp.int32, (1, NN), 1)
        BIGI = jnp.int32(2147483647)

        def body(t, carry):
            d_top, lab_top = carry
            a = d_ref[...]
            m = jnp.min(jnp.min(a, axis=0, keepdims=True))
            sel = jnp.where(a == m, gidx, BIGI)
            idx = jnp.min(jnp.min(sel, axis=0, keepdims=True))
            hit = gidx == idx
            lab = jnp.max(jnp.max(jnp.where(hit, l_ref[...], -1),
                                  axis=0, keepdims=True))
            d_ref[...] = jnp.where(hit, jnp.inf, a)
            d_top = jnp.where(iota64 == t, m, d_top)
            lab_top = jnp.where(iota64 == t, lab, lab_top)
            return d_top, lab_top

        d_top0 = jnp.full((1, NN), jnp.inf, dtype=jnp.float32)
        lab_top0 = jnp.zeros((1, NN), dtype=jnp.int32)
        d_top, lab_top = lax.fori_loop(0, NN, body, (d_top0, lab_top0))

        qv = jnp.float32(1.0) / (-d_top)
        amask = lab_top.reshape(NN, 1) == iota64
        S = jnp.sum(jnp.where(amask, qv, jnp.float32(0.0)),
                    axis=1, keepdims=True)
        rmax = jnp.max(S)
        iota_col = lax.broadcasted_iota(jnp.int32, (NN, 1), 0)
        r = jnp.min(jnp.where(S == rmax, iota_col, jnp.int32(NN)))
        out_ref[...] = (iota64 == r).astype(jnp.float32).reshape(NN)


@jax.jit
def kernel(input, X, y):
    labels = jnp.argmax(y, axis=1).astype(jnp.int32)
    labels = jnp.pad(labels, (0, KP - K)).reshape(NB, 1, BK)
    out = pl.pallas_call(
        _knn_kernel,
        grid=(NB,),
        in_specs=[
            pl.BlockSpec((BK, D), lambda b: (b, 0)),
            pl.BlockSpec((D,), lambda b: (0,)),
            pl.BlockSpec((1, 1, BK), lambda b: (b, 0, 0)),
        ],
        out_specs=pl.BlockSpec((NN,), lambda b: (0,)),
        out_shape=jax.ShapeDtypeStruct((NN,), jnp.float32),
        scratch_shapes=[
            pltpu.VMEM((NB, BK), jnp.float32),
            pltpu.VMEM((NB, BK), jnp.int32),
        ],
    )(X, input, labels)
    return out


# speedup vs baseline: 1.3974x; 1.0306x over previous
import jax
import jax.numpy as jnp
from jax import lax
from jax.experimental import pallas as pl
from jax.experimental.pallas import tpu as pltpu

K = 100000
D = 512
C = 64
NN = 64
BK = 4096
NB = (K + BK - 1) // BK
KP = NB * BK


def _knn_kernel(x_ref, q_ref, lab_ref, out_ref, d_ref, l_ref):
    b = pl.program_id(0)

    xb = x_ref[...]
    q = q_ref[...]
    diff = xb - q[None, :]
    sq = diff * diff
    part = (sq[:, 0:128] + sq[:, 128:256]) + (sq[:, 256:384] + sq[:, 384:512])
    pt = lax.transpose(part, (1, 0))
    dsq = jnp.sum(pt, axis=0, keepdims=True)
    dist = jnp.sqrt(dsq)
    col = lax.broadcasted_iota(jnp.int32, (1, BK), 1)
    valid = (b * BK + col) < K
    dist = jnp.where(valid, dist, jnp.inf)

    d_ref[pl.ds(b, 1), :] = dist
    l_ref[pl.ds(b, 1), :] = lab_ref[...].reshape(1, BK)

    @pl.when(b == NB - 1)
    def _finish():
        row_i = lax.broadcasted_iota(jnp.int32, (NB, BK), 0)
        col_i = lax.broadcasted_iota(jnp.int32, (NB, BK), 1)
        gidx = row_i * BK + col_i
        iota64 = lax.broadcasted_iota(jnp.int32, (1, NN), 1)
        BIGI = jnp.int32(2147483647)

        def body(t, carry):
            d_top, lab_top = carry
            a = d_ref[...]
            m = jnp.min(jnp.min(a, axis=0, keepdims=True))
            sel = jnp.where(a == m, gidx, BIGI)
            idx = jnp.min(jnp.min(sel, axis=0, keepdims=True))
            hit = gidx == idx
            lab = jnp.max(jnp.max(jnp.where(hit, l_ref[...], -1),
                                  axis=0, keepdims=True))
            d_ref[...] = jnp.where(hit, jnp.inf, a)
            d_top = jnp.where(iota64 == t, m, d_top)
            lab_top = jnp.where(iota64 == t, lab, lab_top)
            return d_top, lab_top

        d_top0 = jnp.full((1, NN), jnp.inf, dtype=jnp.float32)
        lab_top0 = jnp.zeros((1, NN), dtype=jnp.int32)
        d_top, lab_top = lax.fori_loop(0, NN, body, (d_top0, lab_top0))

        qv = jnp.float32(1.0) / (-d_top)
        amask = lab_top.reshape(NN, 1) == iota64
        S = jnp.sum(jnp.where(amask, qv, jnp.float32(0.0)),
                    axis=1, keepdims=True)
        rmax = jnp.max(S)
        iota_col = lax.broadcasted_iota(jnp.int32, (NN, 1), 0)
        r = jnp.min(jnp.where(S == rmax, iota_col, jnp.int32(NN)))
        out_ref[...] = (iota64 == r).astype(jnp.float32).reshape(NN)


@jax.jit
def kernel(input, X, y):
    labels = jnp.argmax(y, axis=1).astype(jnp.int32)
    labels = jnp.pad(labels, (0, KP - K)).reshape(NB, 1, BK)
    out = pl.pallas_call(
        _knn_kernel,
        grid=(NB,),
        in_specs=[
            pl.BlockSpec((BK, D), lambda b: (b, 0)),
            pl.BlockSpec((D,), lambda b: (0,)),
            pl.BlockSpec((1, 1, BK), lambda b: (b, 0, 0)),
        ],
        out_specs=pl.BlockSpec((NN,), lambda b: (0,)),
        out_shape=jax.ShapeDtypeStruct((NN,), jnp.float32),
        scratch_shapes=[
            pltpu.VMEM((NB, BK), jnp.float32),
            pltpu.VMEM((NB, BK), jnp.int32),
        ],
    )(X, input, labels)
    return out


# speedup vs baseline: 1.6245x; 1.1625x over previous
import jax
import jax.numpy as jnp
from jax import lax
from jax.experimental import pallas as pl
from jax.experimental.pallas import tpu as pltpu

K = 100000
D = 512
C = 64
NN = 64
BK = 4096
NB = (K + BK - 1) // BK
KP = NB * BK
W = BK // 8


def _knn_kernel(x_ref, q_ref, lab_ref, out_ref, d_ref, l_ref,
                sv_ref, gv_ref, lv_ref, dt_ref, lt_ref):
    b = pl.program_id(0)

    xb = x_ref[...]
    q = q_ref[...]
    diff = xb - q[None, :]
    sq = diff * diff
    part = (sq[:, 0:128] + sq[:, 128:256]) + (sq[:, 256:384] + sq[:, 384:512])
    pt = lax.transpose(part, (1, 0))
    dsq = jnp.sum(pt, axis=0, keepdims=True)
    dist = jnp.sqrt(dsq)
    col1 = lax.broadcasted_iota(jnp.int32, (1, BK), 1)
    valid = (b * BK + col1) < K
    dist = jnp.where(valid, dist, jnp.inf)

    d_ref[pl.ds(b, 1), :] = dist
    l_ref[pl.ds(b, 1), :] = lab_ref[...].reshape(1, BK)

    @pl.when(b == NB - 1)
    def _finish():
        row2 = lax.broadcasted_iota(jnp.int32, (NB, BK), 0)
        iota64 = lax.broadcasted_iota(jnp.int32, (1, NN), 1)
        BIGI = jnp.int32(2147483647)
        INF = jnp.float32(jnp.inf)

        a = d_ref[...]
        lref = l_ref[...]

        def level(cur):
            s = jnp.min(cur, axis=0, keepdims=True)
            r = jnp.min(jnp.where(cur == s, row2, BIGI),
                        axis=0, keepdims=True)
            hitr = row2 == r
            g = r * BK + col1
            l = jnp.min(jnp.where(hitr, lref, BIGI),
                        axis=0, keepdims=True)
            nxt = jnp.where(hitr, INF, cur)
            return s, g, l, nxt

        s1w, g1w, l1w, a2 = level(a)
        s2w, g2w, l2w, a3 = level(a2)
        s3w, g3w, l3w, _ = level(a3)

        for k, (sw, gw, lw) in enumerate([(s1w, g1w, l1w),
                                          (s2w, g2w, l2w),
                                          (s3w, g3w, l3w)]):
            for u in range(8):
                lo, hi = u * W, (u + 1) * W
                sv_ref[8 * k + u : 8 * k + u + 1, :] = sw[0:1, lo:hi]
                gv_ref[8 * k + u : 8 * k + u + 1, :] = gw[0:1, lo:hi]
                lv_ref[8 * k + u : 8 * k + u + 1, :] = lw[0:1, lo:hi]

        s1 = sv_ref[0:8, :]
        s2 = sv_ref[8:16, :]
        s3 = sv_ref[16:24, :]
        g1 = gv_ref[0:8, :]
        g2 = gv_ref[8:16, :]
        g3 = gv_ref[16:24, :]
        l1 = lv_ref[0:8, :]
        l2 = lv_ref[8:16, :]
        l3 = lv_ref[16:24, :]

        def body(t, carry):
            d_top, lab_top, s1, s2, s3, g1, g2, g3, l1, l2, l3 = carry
            m = jnp.min(s1)
            gc = jnp.min(jnp.where(s1 == m, g1, BIGI))
            cm = g1 == gc
            lab = jnp.max(jnp.where(cm, l1, -1))
            s1 = jnp.where(cm, s2, s1)
            s2 = jnp.where(cm, s3, s2)
            s3 = jnp.where(cm, INF, s3)
            g1 = jnp.where(cm, g2, g1)
            g2 = jnp.where(cm, g3, g2)
            g3 = jnp.where(cm, BIGI, g3)
            l1 = jnp.where(cm, l2, l1)
            l2 = jnp.where(cm, l3, l2)
            l3 = jnp.where(cm, -1, l3)
            d_top = jnp.where(iota64 == t, m, d_top)
            lab_top = jnp.where(iota64 == t, lab, lab_top)
            return d_top, lab_top, s1, s2, s3, g1, g2, g3, l1, l2, l3

        d_top0 = jnp.full((1, NN), INF, dtype=jnp.float32)
        lab_top0 = jnp.zeros((1, NN), dtype=jnp.int32)
        res = lax.fori_loop(0, NN, body,
                            (d_top0, lab_top0, s1, s2, s3,
                             g1, g2, g3, l1, l2, l3))
        dt_ref[...] = res[0]
        lt_ref[...] = res[1]

        V = jnp.max(res[0])
        cnt = jnp.sum((a <= V).astype(jnp.int32), axis=0, keepdims=True)
        need_slow = jnp.max(cnt) > 3

        @pl.when(need_slow)
        def _slow():
            gidx = row2 * BK + col1

            def sbody(t, carry):
                d_top, lab_top = carry
                aa = d_ref[...]
                m = jnp.min(jnp.min(aa, axis=0, keepdims=True))
                sel = jnp.where(aa == m, gidx, BIGI)
                idx = jnp.min(jnp.min(sel, axis=0, keepdims=True))
                hit = gidx == idx
                lab = jnp.max(jnp.max(jnp.where(hit, lref, -1),
                                      axis=0, keepdims=True))
                d_ref[...] = jnp.where(hit, INF, aa)
                d_top = jnp.where(iota64 == t, m, d_top)
                lab_top = jnp.where(iota64 == t, lab, lab_top)
                return d_top, lab_top

            dts, lts = lax.fori_loop(0, NN, sbody, (d_top0, lab_top0))
            dt_ref[...] = dts
            lt_ref[...] = lts

        d_top = dt_ref[...]
        lab_top = lt_ref[...]
        qv = jnp.float32(1.0) / (-d_top)
        amask = lab_top.reshape(NN, 1) == iota64
        S = jnp.sum(jnp.where(amask, qv, jnp.float32(0.0)),
                    axis=1, keepdims=True)
        rmax = jnp.max(S)
        iota_col = lax.broadcasted_iota(jnp.int32, (NN, 1), 0)
        r = jnp.min(jnp.where(S == rmax, iota_col, jnp.int32(NN)))
        out_ref[...] = (iota64 == r).astype(jnp.float32).reshape(NN)


@jax.jit
def kernel(input, X, y):
    labels = jnp.argmax(y, axis=1).astype(jnp.int32)
    labels = jnp.pad(labels, (0, KP - K)).reshape(NB, 1, BK)
    out = pl.pallas_call(
        _knn_kernel,
        grid=(NB,),
        in_specs=[
            pl.BlockSpec((BK, D), lambda b: (b, 0)),
            pl.BlockSpec((D,), lambda b: (0,)),
            pl.BlockSpec((1, 1, BK), lambda b: (b, 0, 0)),
        ],
        out_specs=pl.BlockSpec((NN,), lambda b: (0,)),
        out_shape=jax.ShapeDtypeStruct((NN,), jnp.float32),
        scratch_shapes=[
            pltpu.VMEM((NB, BK), jnp.float32),
            pltpu.VMEM((NB, BK), jnp.int32),
            pltpu.VMEM((24, W), jnp.float32),
            pltpu.VMEM((24, W), jnp.int32),
            pltpu.VMEM((24, W), jnp.int32),
            pltpu.VMEM((1, NN), jnp.float32),
            pltpu.VMEM((1, NN), jnp.int32),
        ],
    )(X, input, labels)
    return out


# speedup vs baseline: 1.7268x; 1.0630x over previous
import jax
import jax.numpy as jnp
from jax import lax
from jax.experimental import pallas as pl
from jax.experimental.pallas import tpu as pltpu

K = 100000
D = 512
C = 64
NN = 64
BK = 4096
NB = (K + BK - 1) // BK
KP = NB * BK
W = BK // 8


def _knn_kernel(x_ref, q_ref, lab_ref, out_ref, d_ref, l_ref,
                sv_ref, gv_ref, lv_ref, dt_ref, lt_ref):
    b = pl.program_id(0)

    xb = x_ref[...]
    q = q_ref[...]
    diff = xb - q[None, :]
    sq = diff * diff
    part = (sq[:, 0:128] + sq[:, 128:256]) + (sq[:, 256:384] + sq[:, 384:512])
    pt = lax.transpose(part, (1, 0))
    dsq = jnp.sum(pt, axis=0, keepdims=True)
    dist = jnp.sqrt(dsq)
    col1 = lax.broadcasted_iota(jnp.int32, (1, BK), 1)
    valid = (b * BK + col1) < K
    dist = jnp.where(valid, dist, jnp.inf)

    d_ref[pl.ds(b, 1), :] = dist
    l_ref[pl.ds(b, 1), :] = lab_ref[...].reshape(1, BK)

    @pl.when(b == NB - 1)
    def _finish():
        row2 = lax.broadcasted_iota(jnp.int32, (NB, BK), 0)
        iota64 = lax.broadcasted_iota(jnp.int32, (1, NN), 1)
        BIGI = jnp.int32(2147483647)
        INF = jnp.float32(jnp.inf)

        a = d_ref[...]
        lref = l_ref[...]

        def level(cur):
            s = jnp.min(cur, axis=0, keepdims=True)
            r = jnp.min(jnp.where(cur == s, row2, BIGI),
                        axis=0, keepdims=True)
            hitr = row2 == r
            g = r * BK + col1
            l = jnp.min(jnp.where(hitr, lref, BIGI),
                        axis=0, keepdims=True)
            nxt = jnp.where(hitr, INF, cur)
            return s, g, l, nxt

        s1w, g1w, l1w, a2 = level(a)
        s2w, g2w, l2w, a3 = level(a2)
        s3w, g3w, l3w, _ = level(a3)

        for k, (sw, gw, lw) in enumerate([(s1w, g1w, l1w),
                                          (s2w, g2w, l2w),
                                          (s3w, g3w, l3w)]):
            for u in range(8):
                lo, hi = u * W, (u + 1) * W
                sv_ref[8 * k + u : 8 * k + u + 1, :] = sw[0:1, lo:hi]
                gv_ref[8 * k + u : 8 * k + u + 1, :] = gw[0:1, lo:hi]
                lv_ref[8 * k + u : 8 * k + u + 1, :] = lw[0:1, lo:hi]

        s1 = sv_ref[0:8, :]
        s2 = sv_ref[8:16, :]
        s3 = sv_ref[16:24, :]
        g1 = gv_ref[0:8, :]
        g2 = gv_ref[8:16, :]
        g3 = gv_ref[16:24, :]
        l1 = lv_ref[0:8, :]
        l2 = lv_ref[8:16, :]
        l3 = lv_ref[16:24, :]

        def body(t, carry):
            d_top, lab_top, s1, s2, s3, g1, g2, g3, l1, l2, l3 = carry
            m = jnp.min(s1)
            gc = jnp.min(jnp.where(s1 == m, g1, BIGI))
            cm = g1 == gc
            lab = jnp.max(jnp.where(cm, l1, -1))
            s1 = jnp.where(cm, s2, s1)
            s2 = jnp.where(cm, s3, s2)
            s3 = jnp.where(cm, INF, s3)
            g1 = jnp.where(cm, g2, g1)
            g2 = jnp.where(cm, g3, g2)
            g3 = jnp.where(cm, BIGI, g3)
            l1 = jnp.where(cm, l2, l1)
            l2 = jnp.where(cm, l3, l2)
            l3 = jnp.where(cm, -1, l3)
            d_top = jnp.where(iota64 == t, m, d_top)
            lab_top = jnp.where(iota64 == t, lab, lab_top)
            return d_top, lab_top, s1, s2, s3, g1, g2, g3, l1, l2, l3

        d_top0 = jnp.full((1, NN), INF, dtype=jnp.float32)
        lab_top0 = jnp.zeros((1, NN), dtype=jnp.int32)
        res = lax.fori_loop(0, NN, body,
                            (d_top0, lab_top0, s1, s2, s3,
                             g1, g2, g3, l1, l2, l3))
        dt_ref[...] = res[0]
        lt_ref[...] = res[1]

        V = jnp.max(res[0])
        cnt = jnp.sum((a <= V).astype(jnp.int32), axis=0, keepdims=True)
        need_slow = jnp.max(cnt) > 3

        @pl.when(need_slow)
        def _slow():
            gidx = row2 * BK + col1

            def sbody(t, carry):
                d_top, lab_top = carry
                aa = d_ref[...]
                m = jnp.min(jnp.min(aa, axis=0, keepdims=True))
                sel = jnp.where(aa == m, gidx, BIGI)
                idx = jnp.min(jnp.min(sel, axis=0, keepdims=True))
                hit = gidx == idx
                lab = jnp.max(jnp.max(jnp.where(hit, lref, -1),
                                      axis=0, keepdims=True))
                d_ref[...] = jnp.where(hit, INF, aa)
                d_top = jnp.where(iota64 == t, m, d_top)
                lab_top = jnp.where(iota64 == t, lab, lab_top)
                return d_top, lab_top

            dts, lts = lax.fori_loop(0, NN, sbody, (d_top0, lab_top0))
            dt_ref[...] = dts
            lt_ref[...] = lts

        d_top = dt_ref[...]
        lab_top = lt_ref[...]
        qv = jnp.float32(1.0) / (-d_top)
        amask = lab_top.reshape(NN, 1) == iota64
        S = jnp.sum(jnp.where(amask, qv, jnp.float32(0.0)),
                    axis=1, keepdims=True)
        rmax = jnp.max(S)
        iota_col = lax.broadcasted_iota(jnp.int32, (NN, 1), 0)
        r = jnp.min(jnp.where(S == rmax, iota_col, jnp.int32(NN)))
        out_ref[...] = (iota64 == r).astype(jnp.float32).reshape(NN)


@jax.jit
def kernel(input, X, y):
    labels = jnp.dot(y, jnp.arange(C, dtype=jnp.float32)).astype(jnp.int32)
    labels = jnp.pad(labels, (0, KP - K)).reshape(NB, 1, BK)
    out = pl.pallas_call(
        _knn_kernel,
        grid=(NB,),
        in_specs=[
            pl.BlockSpec((BK, D), lambda b: (b, 0)),
            pl.BlockSpec((D,), lambda b: (0,)),
            pl.BlockSpec((1, 1, BK), lambda b: (b, 0, 0)),
        ],
        out_specs=pl.BlockSpec((NN,), lambda b: (0,)),
        out_shape=jax.ShapeDtypeStruct((NN,), jnp.float32),
        scratch_shapes=[
            pltpu.VMEM((NB, BK), jnp.float32),
            pltpu.VMEM((NB, BK), jnp.int32),
            pltpu.VMEM((24, W), jnp.float32),
            pltpu.VMEM((24, W), jnp.int32),
            pltpu.VMEM((24, W), jnp.int32),
            pltpu.VMEM((1, NN), jnp.float32),
            pltpu.VMEM((1, NN), jnp.int32),
        ],
    )(X, input, labels)
    return out
